# Initial kernel scaffold; baseline (speedup 1.0000x reference)
import jax
import jax.numpy as jnp
from jax.experimental import pallas as pl


def kernel(length, centers, table):
    raise NotImplementedError("write your pallas kernel here")


# speedup vs baseline: 3.5201x; 3.5201x over previous
import functools

import jax
import jax.numpy as jnp
from jax import lax
from jax.experimental import pallas as pl
from jax.experimental.pallas import tpu as pltpu
from jax.experimental.pallas import tpu_sc as plsc

N_TABLE = 32768
D = 128
N_OUT = N_TABLE - 1
NUM_WORKERS = 32
ROWS_PER_W = N_TABLE // NUM_WORKERS
CHUNK = 128
N_CHUNKS = ROWS_PER_W // CHUNK
L = 16


def kernel(length, centers, table):
    params = jnp.concatenate(
        [
            jnp.reshape(jnp.asarray(length, jnp.int32), (1,)),
            centers.astype(jnp.int32).reshape(2),
        ]
    )
    params = jnp.pad(params, (0, L - 3))

    mesh = plsc.VectorSubcoreMesh(core_axis_name="c", subcore_axis_name="s")

    @functools.partial(
        pl.kernel,
        out_type=jax.ShapeDtypeStruct((N_OUT, D), jnp.float32),
        mesh=mesh,
        scratch_types=[
            pltpu.VMEM((L,), jnp.int32),
            pltpu.VMEM((CHUNK,), jnp.int32),
            pltpu.VMEM((CHUNK,), jnp.int32),
            pltpu.VMEM((CHUNK, D), jnp.float32),
            pltpu.SemaphoreType.DMA,
            pltpu.SemaphoreType.DMA,
        ],
        compiler_params=pltpu.CompilerParams(needs_layout_passes=False),
    )
    def run(params_hbm, table_hbm, out_hbm, par_v, gidx_v, oidx_v, rows_v,
            gsem, ssem):
        wid = lax.axis_index("s") * 2 + lax.axis_index("c")
        pltpu.sync_copy(params_hbm, par_v)
        pvec = par_v[...]
        lane = lax.iota(jnp.int32, L)
        len_s = jnp.sum(jnp.where(lane == 0, pvec, 0))
        c1_s = jnp.sum(jnp.where(lane == 2, pvec, 0))
        off_s = len_s - (N_TABLE - 2) - c1_s
        base = wid * ROWS_PER_W
        for c in range(N_CHUNKS):
            row0 = jnp.minimum(base + c * CHUNK, N_OUT - CHUNK)
            for j in range(CHUNK // L):
                orow = row0 + (j * L) + lax.iota(jnp.int32, L)
                ids = jnp.clip(orow + off_s, 0, N_TABLE - 1)
                oidx_v[pl.ds(j * L, L)] = orow
                gidx_v[pl.ds(j * L, L)] = ids
            pltpu.async_copy(table_hbm.at[gidx_v], rows_v, gsem).wait()
            pltpu.async_copy(rows_v, out_hbm.at[oidx_v], ssem).wait()

    return run(params, table)


# speedup vs baseline: 4.1670x; 1.1838x over previous
import functools

import jax
import jax.numpy as jnp
from jax import lax
from jax.experimental import pallas as pl
from jax.experimental.pallas import tpu as pltpu
from jax.experimental.pallas import tpu_sc as plsc

N_TABLE = 32768
D = 128
N_OUT = N_TABLE - 1
NUM_WORKERS = 32
ROWS_PER_W = N_TABLE // NUM_WORKERS
CHUNK = 128
N_CHUNKS = ROWS_PER_W // CHUNK
L = 16


def kernel(length, centers, table):
    params = jnp.concatenate(
        [
            jnp.reshape(jnp.asarray(length, jnp.int32), (1,)),
            centers.astype(jnp.int32).reshape(2),
        ]
    )
    params = jnp.pad(params, (0, L - 3))

    mesh = plsc.VectorSubcoreMesh(core_axis_name="c", subcore_axis_name="s")

    NBUF = 4
    PREF = 2

    @functools.partial(
        pl.kernel,
        out_type=jax.ShapeDtypeStruct((N_OUT, D), jnp.float32),
        mesh=mesh,
        scratch_types=[
            pltpu.VMEM((L,), jnp.int32),
            pltpu.VMEM((N_CHUNKS, CHUNK), jnp.int32),
            pltpu.VMEM((N_CHUNKS, CHUNK), jnp.int32),
            pltpu.VMEM((NBUF, CHUNK, D), jnp.float32),
            pltpu.SemaphoreType.DMA((NBUF,)),
            pltpu.SemaphoreType.DMA((NBUF,)),
        ],
        compiler_params=pltpu.CompilerParams(needs_layout_passes=False),
    )
    def run(params_hbm, table_hbm, out_hbm, par_v, gidx_v, oidx_v, rows_v,
            gsem, ssem):
        wid = lax.axis_index("s") * 2 + lax.axis_index("c")
        pltpu.sync_copy(params_hbm, par_v)
        pvec = par_v[...]
        lane = lax.iota(jnp.int32, L)
        len_s = jnp.sum(jnp.where(lane == 0, pvec, 0))
        c1_s = jnp.sum(jnp.where(lane == 2, pvec, 0))
        off_s = len_s - (N_TABLE - 2) - c1_s
        base = wid * ROWS_PER_W
        for c in range(N_CHUNKS):
            row0 = jnp.minimum(base + c * CHUNK, N_OUT - CHUNK)
            for j in range(CHUNK // L):
                orow = row0 + (j * L) + lax.iota(jnp.int32, L)
                oidx_v[c, pl.ds(j * L, L)] = orow
                gidx_v[c, pl.ds(j * L, L)] = jnp.clip(
                    orow + off_s, 0, N_TABLE - 1)

        def gather(c):
            b = c % NBUF
            return pltpu.async_copy(
                table_hbm.at[gidx_v.at[c]], rows_v.at[b], gsem.at[b])

        def scatter(c):
            b = c % NBUF
            return pltpu.async_copy(
                rows_v.at[b], out_hbm.at[oidx_v.at[c]], ssem.at[b])

        g, s = {}, {}
        for c in range(PREF):
            g[c] = gather(c)
        for c in range(N_CHUNKS):
            g[c].wait()
            s[c] = scatter(c)
            nc = c + PREF
            if nc < N_CHUNKS:
                prev = nc - NBUF
                if prev >= 0:
                    s[prev].wait()
                g[nc] = gather(nc)
        for c in range(N_CHUNKS - NBUF, N_CHUNKS):
            s[c].wait()

    return run(params, table)
